# Initial kernel scaffold; baseline (speedup 1.0000x reference)
import jax
import jax.numpy as jnp
from jax.experimental import pallas as pl


def kernel(x, hidden_states, masks, W_ih, W_hh, b_ih, b_hh):
    raise NotImplementedError("write your pallas kernel here")


# speedup vs baseline: 5.9439x; 5.9439x over previous
import functools

import jax
import jax.numpy as jnp
from jax.experimental import pallas as pl
import jax.experimental.pallas.tpu as pltpu


def _gru_body(xs_ref, ms_ref, wih_ref, whh_ref, bih_ref, bhh_ref, h0_ref,
              out_ref, hfin_ref, gi_ref, mb_ref, h_ref, *,
              chunk, n, h_dim, nblocks):
    i = pl.program_id(0)

    @pl.when(i == 0)
    def _():
        h_ref[...] = h0_ref[...]

    d = xs_ref.shape[2]
    xc = xs_ref[...].reshape(chunk * n, d)
    gi = jnp.dot(xc, wih_ref[...], preferred_element_type=jnp.float32)
    gi_ref[...] = (gi + bih_ref[...]).reshape(chunk, n, 3 * h_dim)
    mb_ref[...] = jnp.broadcast_to(ms_ref[...][:, :, None], (chunk, n, 128))

    whh = whh_ref[...]
    bhh = bhh_ref[...]

    def step(s, h):
        m = mb_ref[s][:, 0:1]
        h = h * m
        gh = jnp.dot(h, whh, preferred_element_type=jnp.float32) + bhh
        gi_s = gi_ref[s]
        r = jax.nn.sigmoid(gi_s[:, :h_dim] + gh[:, :h_dim])
        z = jax.nn.sigmoid(gi_s[:, h_dim:2 * h_dim] + gh[:, h_dim:2 * h_dim])
        ng = jnp.tanh(gi_s[:, 2 * h_dim:] + r * gh[:, 2 * h_dim:])
        h = (1.0 - z) * ng + z * h
        out_ref[s] = h
        return h

    h = jax.lax.fori_loop(0, chunk, step, h_ref[...])
    h_ref[...] = h

    @pl.when(i == nblocks - 1)
    def _():
        hfin_ref[...] = h


def kernel(x, hidden_states, masks, W_ih, W_hh, b_ih, b_hh):
    n = hidden_states.shape[1]
    h_dim = hidden_states.shape[2]
    t = x.shape[0] // n
    d = x.shape[1]

    chunk = 128 if t % 128 == 0 else t
    nblocks = t // chunk

    xs = x.reshape(t, n, d)
    ms = masks.reshape(t, n).astype(jnp.float32)
    h0 = hidden_states[0]
    wih_t = W_ih.T
    whh_t = W_hh.T
    bih2 = b_ih.reshape(1, 3 * h_dim)
    bhh2 = b_hh.reshape(1, 3 * h_dim)

    grid = (nblocks,)
    body = functools.partial(_gru_body, chunk=chunk, n=n, h_dim=h_dim,
                             nblocks=nblocks)
    out, h_final = pl.pallas_call(
        body,
        grid=grid,
        in_specs=[
            pl.BlockSpec((chunk, n, d), lambda i: (i, 0, 0)),
            pl.BlockSpec((chunk, n), lambda i: (i, 0)),
            pl.BlockSpec((d, 3 * h_dim), lambda i: (0, 0)),
            pl.BlockSpec((h_dim, 3 * h_dim), lambda i: (0, 0)),
            pl.BlockSpec((1, 3 * h_dim), lambda i: (0, 0)),
            pl.BlockSpec((1, 3 * h_dim), lambda i: (0, 0)),
            pl.BlockSpec((n, h_dim), lambda i: (0, 0)),
        ],
        out_specs=[
            pl.BlockSpec((chunk, n, h_dim), lambda i: (i, 0, 0)),
            pl.BlockSpec((n, h_dim), lambda i: (0, 0)),
        ],
        out_shape=[
            jax.ShapeDtypeStruct((t, n, h_dim), jnp.float32),
            jax.ShapeDtypeStruct((n, h_dim), jnp.float32),
        ],
        scratch_shapes=[
            pltpu.VMEM((chunk, n, 3 * h_dim), jnp.float32),
            pltpu.VMEM((chunk, n, 128), jnp.float32),
            pltpu.VMEM((n, h_dim), jnp.float32),
        ],
    )(xs, ms, wih_t, whh_t, bih2, bhh2, h0)

    return out.reshape(t * n, h_dim), h_final[None]
